# baseline (speedup 1.0000x reference)
import functools

import jax
import jax.numpy as jnp
from jax import lax
from jax.experimental import pallas as pl
from jax.experimental.pallas import tpu as pltpu
from jax.experimental.pallas import tpu_sc as plsc

N_NODES = 10000
N_EDGES = 320000
HIDDEN = 128

NC = 2
NS = 16
NW = NC * NS
EPW = N_EDGES // NW
CH = 40
NCHUNK = EPW // CH
RING = 5
N_PAD = 10240
RPS = N_PAD // NS
DEGW = 16


def _copy_rows(src_at, dst_at, stage, n_rows):
    done = 0
    while done < n_rows:
        step = min(128, n_rows - done)
        pltpu.sync_copy(src_at(done, step), stage.at[pl.ds(0, step)])
        pltpu.sync_copy(stage.at[pl.ds(0, step)], dst_at(done, step))
        done += step


def _zero_acc(zeros_hbm, zb, acc, base):
    pltpu.sync_copy(zeros_hbm, zb)
    done = 0
    while done < RPS:
        step = min(128, RPS - done)
        pltpu.sync_copy(zb.at[pl.ds(0, step)], acc.at[pl.ds(base + done, step)])
        done += step


def _deg_body(dst_hbm, ones_hbm, zeros_hbm, deg_out, didx, ones_v, zb, acc):
    c = lax.axis_index("c")
    s = lax.axis_index("s")
    wid = c * NS + s
    base = s * RPS

    pltpu.sync_copy(ones_hbm, ones_v)
    _zero_acc(zeros_hbm, zb, acc, base)
    plsc.subcore_barrier()

    def body(j, carry):
        pltpu.sync_copy(dst_hbm.at[pl.ds(wid * EPW + j * CH, CH)], didx)
        pltpu.sync_copy(ones_v, acc.at[didx], add=True)
        return carry

    lax.fori_loop(0, NCHUNK, body, 0)
    plsc.subcore_barrier()

    _copy_rows(lambda o, n: acc.at[pl.ds(base + o, n)],
               lambda o, n: deg_out.at[c, pl.ds(base + o, n)],
               zb, RPS)


def _agg_body(hs_hbm, src_hbm, dst_hbm, zeros_hbm, agg_out,
              s0, s1, s2, s3, s4, d0, d1, d2, d3, d4,
              r0, r1, r2, r3, r4, zb, acc, gs0, gs1, gs2, gs3, gs4):
    sidx = [s0, s1, s2, s3, s4]
    didx = [d0, d1, d2, d3, d4]
    rows = [r0, r1, r2, r3, r4]
    gsem = [gs0, gs1, gs2, gs3, gs4]
    c = lax.axis_index("c")
    s = lax.axis_index("s")
    wid = c * NS + s
    base = s * RPS

    _zero_acc(zeros_hbm, zb, acc, base)

    def load(p, j):
        eb = wid * EPW + j * CH
        pltpu.sync_copy(src_hbm.at[pl.ds(eb, CH)], sidx[p])
        pltpu.sync_copy(dst_hbm.at[pl.ds(eb, CH)], didx[p])

    def gat(p):
        return pltpu.make_async_copy(hs_hbm.at[sidx[p]], rows[p], gsem[p])

    for p in range(RING):
        load(p, p)
    plsc.subcore_barrier()
    for p in range(RING):
        gat(p).start()

    def body(i, carry):
        for p in range(RING):
            gat(p).wait()
            pltpu.sync_copy(rows[p], acc.at[didx[p]], add=True)
            load(p, (i + 1) * RING + p)
            gat(p).start()
        return carry

    lax.fori_loop(0, NCHUNK // RING - 1, body, 0)
    for p in range(RING):
        gat(p).wait()
        pltpu.sync_copy(rows[p], acc.at[didx[p]], add=True)
    plsc.subcore_barrier()

    _copy_rows(lambda o, n: acc.at[pl.ds(base + o, n)],
               lambda o, n: agg_out.at[c, pl.ds(base + o, n)],
               zb, RPS)


@functools.cache
def _sc_calls():
    mesh = plsc.VectorSubcoreMesh(core_axis_name="c", subcore_axis_name="s",
                                  num_cores=NC, num_subcores=NS)
    deg_call = pl.kernel(
        _deg_body,
        out_type=jax.ShapeDtypeStruct((NC, N_PAD, DEGW), jnp.float32),
        mesh=mesh,
        scratch_types=[
            pltpu.VMEM((CH,), jnp.int32),
            pltpu.VMEM((CH, DEGW), jnp.float32),
            pltpu.VMEM((128, DEGW), jnp.float32),
            pltpu.VMEM_SHARED((N_PAD, DEGW), jnp.float32),
        ],
    )
    agg_call = pl.kernel(
        _agg_body,
        out_type=jax.ShapeDtypeStruct((NC, N_PAD, HIDDEN), jnp.float32),
        mesh=mesh,
        scratch_types=[pltpu.VMEM((CH,), jnp.int32)] * (2 * RING)
        + [pltpu.VMEM((CH, HIDDEN), jnp.float32)] * RING + [
            pltpu.VMEM((128, HIDDEN), jnp.float32),
            pltpu.VMEM_SHARED((N_PAD, HIDDEN), jnp.float32),
        ] + [pltpu.SemaphoreType.DMA] * RING,
    )
    return deg_call, agg_call


def _tc_prep_body(deg_ref, x_ref, w_ref, dinv_ref, hs_ref):
    deg = deg_ref[0, :N_NODES, 0:1] + deg_ref[1, :N_NODES, 0:1] + 1.0
    dinv = lax.rsqrt(deg)
    dinv_ref[...] = dinv
    h = jnp.dot(x_ref[...], w_ref[...], preferred_element_type=jnp.float32)
    hs_ref[...] = h * dinv


def _tc_mid_body(agg_ref, hs_ref, dinv_ref, b_ref, w_ref, out_ref):
    dinv = dinv_ref[...]
    h = dinv * (agg_ref[0, :N_NODES] + agg_ref[1, :N_NODES] + hs_ref[...]) + b_ref[...]
    h = jnp.maximum(h, 0.0)
    out_ref[...] = jnp.dot(h, w_ref[...],
                           preferred_element_type=jnp.float32) * dinv


def _tc_out_body(agg_ref, hs_ref, dinv_ref, b_ref, out_ref):
    h = dinv_ref[...] * (agg_ref[0, :N_NODES] + agg_ref[1, :N_NODES] + hs_ref[...]) + b_ref[...]
    out_ref[...] = jnp.maximum(h, 0.0)


_tc_prep = pl.pallas_call(
    _tc_prep_body,
    out_shape=(jax.ShapeDtypeStruct((N_NODES, 1), jnp.float32),
               jax.ShapeDtypeStruct((N_NODES, HIDDEN), jnp.float32)),
)

_tc_mid = pl.pallas_call(
    _tc_mid_body,
    out_shape=jax.ShapeDtypeStruct((N_NODES, HIDDEN), jnp.float32),
)

_tc_out = pl.pallas_call(
    _tc_out_body,
    out_shape=jax.ShapeDtypeStruct((N_NODES, HIDDEN), jnp.float32),
)


@jax.jit
def _run(x, edge_index, W1, b1, W2, b2):
    src = edge_index[0].astype(jnp.int32)
    dst = edge_index[1].astype(jnp.int32)
    ones = jnp.ones((CH, DEGW), jnp.float32)
    zeros_d = jnp.zeros((128, DEGW), jnp.float32)
    zeros_h = jnp.zeros((128, HIDDEN), jnp.float32)

    deg_call, agg_call = _sc_calls()
    deg = deg_call(dst, ones, zeros_d)
    dinv, hs1 = _tc_prep(deg, x, W1)
    agg1 = agg_call(hs1, src, dst, zeros_h)
    hs2 = _tc_mid(agg1, hs1, dinv, b1.reshape(1, HIDDEN), W2)
    agg2 = agg_call(hs2, src, dst, zeros_h)
    return _tc_out(agg2, hs2, dinv, b2.reshape(1, HIDDEN))


def kernel(x, edge_index, W1, b1, W2, b2):
    return _run(x, edge_index, W1, b1, W2, b2)


# speedup vs baseline: 1.5191x; 1.5191x over previous
import functools

import jax
import jax.numpy as jnp
from jax import lax
from jax.experimental import pallas as pl
from jax.experimental.pallas import tpu as pltpu
from jax.experimental.pallas import tpu_sc as plsc

N_NODES = 10000
N_EDGES = 320000
HIDDEN = 128

NC = 2
NS = 16
NW = NC * NS
EPW = N_EDGES // NW
CH = 80
NCHUNK = EPW // CH
RING = 2
N_PAD = 10240
RPS = N_PAD // NS
DEGW = 16


def _copy_rows(src_at, dst_at, stage, n_rows):
    done = 0
    while done < n_rows:
        step = min(128, n_rows - done)
        pltpu.sync_copy(src_at(done, step), stage.at[pl.ds(0, step)])
        pltpu.sync_copy(stage.at[pl.ds(0, step)], dst_at(done, step))
        done += step


def _zero_acc(zeros_hbm, zb, acc, base):
    pltpu.sync_copy(zeros_hbm, zb)
    done = 0
    while done < RPS:
        step = min(128, RPS - done)
        pltpu.sync_copy(zb.at[pl.ds(0, step)], acc.at[pl.ds(base + done, step)])
        done += step


def _deg_body(dst_hbm, ones_hbm, zeros_hbm, deg_out, didx, ones_v, zb, acc):
    c = lax.axis_index("c")
    s = lax.axis_index("s")
    wid = c * NS + s
    base = s * RPS

    pltpu.sync_copy(ones_hbm, ones_v)
    _zero_acc(zeros_hbm, zb, acc, base)
    plsc.subcore_barrier()

    def body(j, carry):
        pltpu.sync_copy(dst_hbm.at[pl.ds(wid * EPW + j * CH, CH)], didx)
        pltpu.sync_copy(ones_v, acc.at[didx], add=True)
        return carry

    lax.fori_loop(0, NCHUNK, body, 0)
    plsc.subcore_barrier()

    _copy_rows(lambda o, n: acc.at[pl.ds(base + o, n)],
               lambda o, n: deg_out.at[c, pl.ds(base + o, n)],
               zb, RPS)


def _agg_body(hs_hbm, src_hbm, dst_hbm, zeros_hbm, agg_out,
              s0, s1, d0, d1, r0, r1, zb, acc, gs0, gs1):
    sidx = [s0, s1]
    didx = [d0, d1]
    rows = [r0, r1]
    gsem = [gs0, gs1]
    c = lax.axis_index("c")
    s = lax.axis_index("s")
    wid = c * NS + s
    base = s * RPS

    _zero_acc(zeros_hbm, zb, acc, base)

    def load(p, j):
        eb = wid * EPW + j * CH
        pltpu.sync_copy(src_hbm.at[pl.ds(eb, CH)], sidx[p])
        pltpu.sync_copy(dst_hbm.at[pl.ds(eb, CH)], didx[p])

    def gat(p):
        return pltpu.make_async_copy(hs_hbm.at[sidx[p]], rows[p], gsem[p])

    def slot(p, j, start_next):
        gat(p).wait()
        pltpu.sync_copy(rows[p], acc.at[didx[p]], add=True)
        if start_next:
            load(p, j)
            gat(p).start()

    load(0, 0)
    load(1, 1)
    plsc.subcore_barrier()
    gat(0).start()
    gat(1).start()

    def body(i, carry):
        j = i * 2
        slot(0, j + 2, True)
        slot(1, j + 3, True)
        return carry

    lax.fori_loop(0, NCHUNK // 2 - 1, body, 0)
    slot(0, NCHUNK - 1, True)
    slot(1, 0, False)
    slot(0, 0, False)
    plsc.subcore_barrier()

    _copy_rows(lambda o, n: acc.at[pl.ds(base + o, n)],
               lambda o, n: agg_out.at[c, pl.ds(base + o, n)],
               zb, RPS)


@functools.cache
def _sc_calls():
    mesh = plsc.VectorSubcoreMesh(core_axis_name="c", subcore_axis_name="s",
                                  num_cores=NC, num_subcores=NS)
    deg_call = pl.kernel(
        _deg_body,
        out_type=jax.ShapeDtypeStruct((NC, N_PAD, DEGW), jnp.float32),
        mesh=mesh,
        scratch_types=[
            pltpu.VMEM((CH,), jnp.int32),
            pltpu.VMEM((CH, DEGW), jnp.float32),
            pltpu.VMEM((128, DEGW), jnp.float32),
            pltpu.VMEM_SHARED((N_PAD, DEGW), jnp.float32),
        ],
    )
    agg_call = pl.kernel(
        _agg_body,
        out_type=jax.ShapeDtypeStruct((NC, N_PAD, HIDDEN), jnp.float32),
        mesh=mesh,
        scratch_types=[pltpu.VMEM((CH,), jnp.int32)] * 4
        + [pltpu.VMEM((CH, HIDDEN), jnp.float32)] * 2 + [
            pltpu.VMEM((128, HIDDEN), jnp.float32),
            pltpu.VMEM_SHARED((N_PAD, HIDDEN), jnp.float32),
        ] + [pltpu.SemaphoreType.DMA] * 2,
    )
    return deg_call, agg_call


def _tc_prep_body(deg_ref, x_ref, w_ref, dinv_ref, hs_ref):
    deg = deg_ref[0, :N_NODES, 0:1] + deg_ref[1, :N_NODES, 0:1] + 1.0
    dinv = lax.rsqrt(deg)
    dinv_ref[...] = dinv
    h = jnp.dot(x_ref[...], w_ref[...], preferred_element_type=jnp.float32)
    hs_ref[...] = h * dinv


def _tc_mid_body(agg_ref, hs_ref, dinv_ref, b_ref, w_ref, out_ref):
    dinv = dinv_ref[...]
    h = dinv * (agg_ref[0, :N_NODES] + agg_ref[1, :N_NODES] + hs_ref[...]) + b_ref[...]
    h = jnp.maximum(h, 0.0)
    out_ref[...] = jnp.dot(h, w_ref[...],
                           preferred_element_type=jnp.float32) * dinv


def _tc_out_body(agg_ref, hs_ref, dinv_ref, b_ref, out_ref):
    h = dinv_ref[...] * (agg_ref[0, :N_NODES] + agg_ref[1, :N_NODES] + hs_ref[...]) + b_ref[...]
    out_ref[...] = jnp.maximum(h, 0.0)


_tc_prep = pl.pallas_call(
    _tc_prep_body,
    out_shape=(jax.ShapeDtypeStruct((N_NODES, 1), jnp.float32),
               jax.ShapeDtypeStruct((N_NODES, HIDDEN), jnp.float32)),
)

_tc_mid = pl.pallas_call(
    _tc_mid_body,
    out_shape=jax.ShapeDtypeStruct((N_NODES, HIDDEN), jnp.float32),
)

_tc_out = pl.pallas_call(
    _tc_out_body,
    out_shape=jax.ShapeDtypeStruct((N_NODES, HIDDEN), jnp.float32),
)


@jax.jit
def _run(x, edge_index, W1, b1, W2, b2):
    src = edge_index[0].astype(jnp.int32)
    dst = edge_index[1].astype(jnp.int32)
    ones = jnp.ones((CH, DEGW), jnp.float32)
    zeros_d = jnp.zeros((128, DEGW), jnp.float32)
    zeros_h = jnp.zeros((128, HIDDEN), jnp.float32)

    deg_call, agg_call = _sc_calls()
    deg = deg_call(dst, ones, zeros_d)
    dinv, hs1 = _tc_prep(deg, x, W1)
    agg1 = agg_call(hs1, src, dst, zeros_h)
    hs2 = _tc_mid(agg1, hs1, dinv, b1.reshape(1, HIDDEN), W2)
    agg2 = agg_call(hs2, src, dst, zeros_h)
    return _tc_out(agg2, hs2, dinv, b2.reshape(1, HIDDEN))


def kernel(x, edge_index, W1, b1, W2, b2):
    return _run(x, edge_index, W1, b1, W2, b2)
